# Initial kernel scaffold; baseline (speedup 1.0000x reference)
import jax
import jax.numpy as jnp
from jax.experimental import pallas as pl


def kernel(x_user, x_item, edge_index_user_to_item, edge_index_item_rev_to_user, batch_user, batch_item, params):
    raise NotImplementedError("write your pallas kernel here")


# speedup vs baseline: 1.8290x; 1.8290x over previous
import functools

import jax
import jax.numpy as jnp
from jax.experimental import pallas as pl

N_USER = 50000
N_ITEM = 50000
G = 256
H = 128
OUT = 64

_ROW_BLK = 2000


def _dense_body(x_ref, wsrc_ref, wdst_ref, attsrc_ref, attdst_ref, h_ref, a_ref):
    x = x_ref[...]
    h = jnp.dot(x, wsrc_ref[...], preferred_element_type=jnp.float32)
    h_ref[...] = h
    a_s = jnp.dot(h, attsrc_ref[...], preferred_element_type=jnp.float32)
    vd = jnp.dot(wdst_ref[...], attdst_ref[...], preferred_element_type=jnp.float32)
    a_d = jnp.dot(x, vd, preferred_element_type=jnp.float32)
    a_ref[:, 0] = a_s
    a_ref[:, 1] = a_d


def _dense(x, w_src, att_src, w_dst_o, att_dst_o):
    n = x.shape[0]
    grid = (n // _ROW_BLK,)
    return pl.pallas_call(
        _dense_body,
        grid=grid,
        in_specs=[
            pl.BlockSpec((_ROW_BLK, H), lambda i: (i, 0)),
            pl.BlockSpec((H, H), lambda i: (0, 0)),
            pl.BlockSpec((H, H), lambda i: (0, 0)),
            pl.BlockSpec((H,), lambda i: (0,)),
            pl.BlockSpec((H,), lambda i: (0,)),
        ],
        out_specs=[
            pl.BlockSpec((_ROW_BLK, H), lambda i: (i, 0)),
            pl.BlockSpec((_ROW_BLK, 2), lambda i: (i, 0)),
        ],
        out_shape=[
            jax.ShapeDtypeStruct((n, H), jnp.float32),
            jax.ShapeDtypeStruct((n, 2), jnp.float32),
        ],
    )(x, w_src, w_dst_o, att_src, att_dst_o)


def _leaky_relu(x, slope=0.2):
    return jnp.where(x > 0, x, slope * x)


def _gat_conv(hs, a_src, a_dst, edge_index, bias, n_dst):
    src = edge_index[0]
    dst = edge_index[1]
    ex = jnp.exp(_leaky_relu(a_src[src] + a_dst[dst]))
    denom = jax.ops.segment_sum(ex, dst, num_segments=n_dst)
    msg = hs[src] * ex[:, None]
    out = jax.ops.segment_sum(msg, dst, num_segments=n_dst)
    return out / (denom[:, None] + 1e-16) + bias


def kernel(x_user, x_item, edge_index_user_to_item, edge_index_item_rev_to_user,
           batch_user, batch_item, params):
    xu, xi = x_user, x_item
    for layer in (0, 1):
        pu = params['l%d_u2i' % layer]
        pi = params['l%d_i2u' % layer]
        hs_u, a_u = _dense(xu, pu['W_src'], pu['att_src'], pi['W_dst'], pi['att_dst'])
        hs_i, a_i = _dense(xi, pi['W_src'], pi['att_src'], pu['W_dst'], pu['att_dst'])
        new_item = _gat_conv(hs_u, a_u[:, 0], a_i[:, 1], edge_index_user_to_item,
                             pu['bias'], N_ITEM)
        new_user = _gat_conv(hs_i, a_i[:, 0], a_u[:, 1], edge_index_item_rev_to_user,
                             pi['bias'], N_USER)
        xu = jax.nn.relu(new_user)
        xi = jax.nn.relu(new_item)
    pu = jax.ops.segment_max(xu, batch_user, num_segments=G)
    pu = jnp.where(jnp.isfinite(pu), pu, 0.0)
    pi = jax.ops.segment_max(xi, batch_item, num_segments=G)
    pi = jnp.where(jnp.isfinite(pi), pi, 0.0)
    out_user = pu @ params['lin_W'] + params['lin_b']
    out_item = pi @ params['lin_W'] + params['lin_b']
    return (out_user, out_item)
